# baseline (speedup 1.0000x reference)
import functools

import numpy as np
import jax
import jax.numpy as jnp
from jax import lax
from jax.experimental import pallas as pl
from jax.experimental.pallas import tpu as pltpu
from jax.experimental.pallas import tpu_sc as plsc

_GRID = 128
_C = 28
_CP = 32
_NW = 32
_B = 128

_SH_C0 = 0.5 * np.sqrt(1.0 / np.pi)
_SH_C1 = np.sqrt(3.0 / (4 * np.pi))
_SH_C2_XY = 0.5 * np.sqrt(15.0 / (4 * np.pi))
_SH_C2_Z2 = 0.5 * np.sqrt(5.0 / (16 * np.pi))
_SH_C2_X2Y2 = 0.5 * np.sqrt(15.0 / (16 * np.pi))


def _axis_coords(p):
    i = jnp.clip(p * 15.875 + 63.5, 0.0, 127.0)
    i0 = i.astype(jnp.int32)
    w = i - i0.astype(jnp.float32)
    i1 = jnp.minimum(i0 + 1, _GRID - 1)
    return i0, i1, w


def _prep_body(pts_ref, idx_ref):
    px = pts_ref[0, :]
    py = pts_ref[1, :]
    pz = pts_ref[2, :]
    x0, x1, _ = _axis_coords(px)
    y0, y1, _ = _axis_coords(py)
    z0, z1, _ = _axis_coords(pz)
    c = 0
    for zz in (z0, z1):
        for yy in (y0, y1):
            for xx in (x0, x1):
                idx_ref[c, :] = (zz * _GRID + yy) * _GRID + xx
                c += 1


def _tc_prep(pts_t):
    n = pts_t.shape[1]
    nb = 2048
    return pl.pallas_call(
        _prep_body,
        grid=(n // nb,),
        in_specs=[pl.BlockSpec((3, nb), lambda i: (0, i))],
        out_specs=pl.BlockSpec((8, nb), lambda i: (0, i)),
        out_shape=jax.ShapeDtypeStruct((8, n), jnp.int32),
    )(pts_t)


def _sc_gather(idx8, table):
    n = idx8.shape[1]
    pw = n // _NW
    nch = pw // _B
    mesh = plsc.VectorSubcoreMesh(core_axis_name="c", subcore_axis_name="s")

    @functools.partial(
        pl.kernel,
        mesh=mesh,
        compiler_params=pltpu.CompilerParams(use_tc_tiling_on_sc=False),
        out_type=jax.ShapeDtypeStruct((8, n, _CP), jnp.float32),
        scratch_types=[
            pltpu.VMEM((8, _B), jnp.int32),
            pltpu.VMEM((8, _B, _CP), jnp.float32),
            pltpu.SemaphoreType.DMA,
        ],
    )
    def k(idx_hbm, table_hbm, out_hbm, idx_v, rows_v, sem):
        wid = lax.axis_index("s") * 2 + lax.axis_index("c")
        base_w = wid * pw

        def body(ci, carry):
            base = base_w + ci * _B
            for c in range(8):
                pltpu.sync_copy(idx_hbm.at[c, pl.ds(base, _B)], idx_v.at[c])
            cps = [
                pltpu.async_copy(table_hbm.at[idx_v.at[c]], rows_v.at[c], sem)
                for c in range(8)
            ]
            for cp in cps:
                cp.wait()
            for c in range(8):
                pltpu.sync_copy(rows_v.at[c], out_hbm.at[c, pl.ds(base, _B), :])
            return carry

        lax.fori_loop(0, nch, body, 0)

    return k(idx8, table)


def _shade_body(pts_ref, cor_ref, rgb_ref, sig_ref):
    px = pts_ref[:, 0:1]
    py = pts_ref[:, 1:2]
    pz = pts_ref[:, 2:3]
    _, _, wx = _axis_coords(px)
    _, _, wy = _axis_coords(py)
    _, _, wz = _axis_coords(pz)
    ox, oy, oz = 1.0 - wx, 1.0 - wy, 1.0 - wz
    wgt = (
        oz * oy * ox, oz * oy * wx, oz * wy * ox, oz * wy * wx,
        wz * oy * ox, wz * oy * wx, wz * wy * ox, wz * wy * wx,
    )
    feats = wgt[0] * cor_ref[0]
    for c in range(1, 8):
        feats = feats + wgt[c] * cor_ref[c]

    rn = lax.rsqrt(px * px + py * py + pz * pz)
    x, y, z = px * rn, py * rn, pz * rn
    basis = [
        _SH_C0 * jnp.ones_like(x),
        _SH_C1 * y, _SH_C1 * z, _SH_C1 * x,
        _SH_C2_XY * (x * y), _SH_C2_XY * (y * z),
        _SH_C2_Z2 * (3.0 * z * z - 1.0),
        _SH_C2_XY * (x * z), _SH_C2_X2Y2 * (x * x - y * y),
    ]
    bexp = jnp.concatenate([jnp.ones_like(x)] + basis * 3 + [jnp.ones_like(x)] * 4, axis=1)
    jj = lax.broadcasted_iota(jnp.int32, (_CP, 4), 0)
    rr = lax.broadcasted_iota(jnp.int32, (_CP, 4), 1)
    pmat = jnp.where(
        ((jj == 0) & (rr == 0)) | ((jj >= 1) & ((jj - 1) // 9 == rr - 1)),
        1.0, 0.0,
    )
    pre = jnp.dot(feats * bexp, pmat, precision=lax.Precision.HIGHEST)
    d = pre[:, 0:1]
    rgbp = pre[:, 1:4]
    rgb_ref[...] = 1.0 / (1.0 + jnp.exp(-rgbp))
    sig_ref[...] = jnp.maximum(d, 0.0) + jnp.log1p(jnp.exp(-jnp.abs(d)))


def _tc_shade(points, corners):
    n = points.shape[0]
    nb = 1024
    return pl.pallas_call(
        _shade_body,
        grid=(n // nb,),
        in_specs=[
            pl.BlockSpec((nb, 3), lambda i: (i, 0)),
            pl.BlockSpec((8, nb, _CP), lambda i: (0, i, 0)),
        ],
        out_specs=[
            pl.BlockSpec((nb, 3), lambda i: (i, 0)),
            pl.BlockSpec((nb, 1), lambda i: (i, 0)),
        ],
        out_shape=[
            jax.ShapeDtypeStruct((n, 3), jnp.float32),
            jax.ShapeDtypeStruct((n, 1), jnp.float32),
        ],
    )(points, corners)


def kernel(points, voxels):
    table = jnp.pad(voxels.reshape(-1, _C), ((0, 0), (0, _CP - _C)))
    pts_t = points.T
    idx8 = _tc_prep(pts_t)
    corners = _sc_gather(idx8, table)
    rgb, sigma = _tc_shade(points, corners)
    return rgb, sigma


# speedup vs baseline: 2.3670x; 2.3670x over previous
import functools

import numpy as np
import jax
import jax.numpy as jnp
from jax import lax
from jax.experimental import pallas as pl
from jax.experimental.pallas import tpu as pltpu
from jax.experimental.pallas import tpu_sc as plsc

_GRID = 128
_C = 28
_CP = 32
_NW = 32
_B = 128

_SH_C0 = 0.5 * np.sqrt(1.0 / np.pi)
_SH_C1 = np.sqrt(3.0 / (4 * np.pi))
_SH_C2_XY = 0.5 * np.sqrt(15.0 / (4 * np.pi))
_SH_C2_Z2 = 0.5 * np.sqrt(5.0 / (16 * np.pi))
_SH_C2_X2Y2 = 0.5 * np.sqrt(15.0 / (16 * np.pi))


def _axis_coords(p):
    i = jnp.minimum(jnp.maximum(p * 15.875 + 63.5, 0.0), 127.0)
    i0 = i.astype(jnp.int32)
    i0 = jnp.where(i0.astype(jnp.float32) > i, i0 - 1, i0)
    w = i - i0.astype(jnp.float32)
    i1 = jnp.minimum(i0 + 1, _GRID - 1)
    return i0, i1, w


def _sc_fused(pts_t, table):
    n = pts_t.shape[1]
    pw = n // _NW
    nch = pw // _B
    mesh = plsc.VectorSubcoreMesh(core_axis_name="c", subcore_axis_name="s")

    @functools.partial(
        pl.kernel,
        mesh=mesh,
        compiler_params=pltpu.CompilerParams(use_tc_tiling_on_sc=False),
        out_type=jax.ShapeDtypeStruct((n, _CP), jnp.float32),
        scratch_types=[
            pltpu.VMEM((3, _B), jnp.float32),
            pltpu.VMEM((2, 8, _B), jnp.int32),
            pltpu.VMEM((2, 8, _B), jnp.float32),
            pltpu.VMEM((2, 8, _B, _CP), jnp.float32),
            pltpu.VMEM((_B, _CP), jnp.float32),
            pltpu.SemaphoreType.DMA,
            pltpu.SemaphoreType.DMA,
        ],
    )
    def k(pts_hbm, table_hbm, out_hbm, pts_v, idx_v, w_v, rows_v, ft_v,
          sem0, sem1):
        wid = lax.axis_index("s") * 2 + lax.axis_index("c")
        base_w = wid * pw
        sems = (sem0, sem1)

        def stage_fire(ci, b):
            base = base_w + ci * _B
            for d in range(3):
                pltpu.sync_copy(pts_hbm.at[d, pl.ds(base, _B)], pts_v.at[d])
            for g in range(_B // 16):
                s = g * 16
                x0, x1, wx = _axis_coords(pts_v[0, pl.ds(s, 16)])
                y0, y1, wy = _axis_coords(pts_v[1, pl.ds(s, 16)])
                z0, z1, wz = _axis_coords(pts_v[2, pl.ds(s, 16)])
                ox, oy, oz = 1.0 - wx, 1.0 - wy, 1.0 - wz
                b00 = (z0 * _GRID + y0) * _GRID
                b01 = (z0 * _GRID + y1) * _GRID
                b10 = (z1 * _GRID + y0) * _GRID
                b11 = (z1 * _GRID + y1) * _GRID
                corn = (
                    (b00 + x0, oz * oy * ox), (b00 + x1, oz * oy * wx),
                    (b01 + x0, oz * wy * ox), (b01 + x1, oz * wy * wx),
                    (b10 + x0, wz * oy * ox), (b10 + x1, wz * oy * wx),
                    (b11 + x0, wz * wy * ox), (b11 + x1, wz * wy * wx),
                )
                for c, (iv, wv) in enumerate(corn):
                    idx_v[b, c, pl.ds(s, 16)] = iv
                    w_v[b, c, pl.ds(s, 16)] = wv
            for c in range(8):
                pltpu.async_copy(
                    table_hbm.at[idx_v.at[b, c]], rows_v.at[b, c], sems[b])

        def stage_blend(ci, b):
            base = base_w + ci * _B
            for c in range(8):
                pltpu.make_async_copy(
                    table_hbm.at[idx_v.at[b, c]], rows_v.at[b, c],
                    sems[b]).wait()
            for g in range(_B // 16):
                s = g * 16
                wrows = [w_v[b, c, pl.ds(s, 16)] for c in range(8)]

                def ptl(l, carry2, s=s, wrows=wrows):
                    p = s + l
                    lsel = jnp.full((16,), l, jnp.int32)
                    wbs = [jnp.take(wrows[c], lsel) for c in range(8)]
                    for off in (0, 16):
                        acc = wbs[0] * rows_v[b, 0, p, pl.ds(off, 16)]
                        for c in range(1, 8):
                            acc = acc + wbs[c] * rows_v[b, c, p, pl.ds(off, 16)]
                        ft_v[p, pl.ds(off, 16)] = acc
                    return carry2

                lax.fori_loop(0, 16, ptl, 0)
            pltpu.sync_copy(ft_v, out_hbm.at[pl.ds(base, _B), :])

        stage_fire(0, 0)

        def pair(k2, carry):
            ci0 = 2 * k2
            stage_fire(ci0 + 1, 1)
            stage_blend(ci0, 0)

            @pl.when(ci0 + 2 < nch)
            def _():
                stage_fire(ci0 + 2, 0)

            stage_blend(ci0 + 1, 1)
            return carry

        lax.fori_loop(0, nch // 2, pair, 0)

    return k(pts_t, table)


def _shade_body(pts_ref, ft_ref, rgb_ref, sig_ref):
    ftt = jnp.transpose(ft_ref[...], (1, 0))
    px = pts_ref[0, :]
    py = pts_ref[1, :]
    pz = pts_ref[2, :]
    rn = lax.rsqrt(px * px + py * py + pz * pz)
    x, y, z = px * rn, py * rn, pz * rn
    basis = [
        _SH_C0 * jnp.ones_like(x),
        _SH_C1 * y, _SH_C1 * z, _SH_C1 * x,
        _SH_C2_XY * (x * y), _SH_C2_XY * (y * z),
        _SH_C2_Z2 * (3.0 * z * z - 1.0),
        _SH_C2_XY * (x * z), _SH_C2_X2Y2 * (x * x - y * y),
    ]
    for r in range(3):
        acc = basis[0] * ftt[1 + 9 * r, :]
        for kk in range(1, 9):
            acc = acc + basis[kk] * ftt[1 + 9 * r + kk, :]
        rgb_ref[r, :] = 1.0 / (1.0 + jnp.exp(-acc))
    d = ftt[0, :]
    sig_ref[0, :] = jnp.maximum(d, 0.0) + jnp.log1p(jnp.exp(-jnp.abs(d)))


def _tc_shade(pts_t, ft):
    n = pts_t.shape[1]
    nb = 1024
    return pl.pallas_call(
        _shade_body,
        grid=(n // nb,),
        in_specs=[
            pl.BlockSpec((3, nb), lambda i: (0, i)),
            pl.BlockSpec((nb, _CP), lambda i: (i, 0)),
        ],
        out_specs=[
            pl.BlockSpec((3, nb), lambda i: (0, i)),
            pl.BlockSpec((1, nb), lambda i: (0, i)),
        ],
        out_shape=[
            jax.ShapeDtypeStruct((3, n), jnp.float32),
            jax.ShapeDtypeStruct((1, n), jnp.float32),
        ],
    )(pts_t, ft)


def kernel(points, voxels):
    n = points.shape[0]
    table = jnp.pad(voxels.reshape(-1, _C), ((0, 0), (0, _CP - _C)))
    pts_t = points.T
    ft = _sc_fused(pts_t, table)
    rgb_t, sig_t = _tc_shade(pts_t, ft)
    rgb = rgb_t.T
    sigma = sig_t.reshape(n)[:, None]
    return rgb, sigma


# speedup vs baseline: 3.9603x; 1.6731x over previous
import functools

import numpy as np
import jax
import jax.numpy as jnp
from jax import lax
from jax.experimental import pallas as pl
from jax.experimental.pallas import tpu as pltpu
from jax.experimental.pallas import tpu_sc as plsc

_GRID = 128
_C = 28
_CP = 32
_W = _CP // 2
_NW = 32
_B = 128

_SH_C0 = 0.5 * np.sqrt(1.0 / np.pi)
_SH_C1 = np.sqrt(3.0 / (4 * np.pi))
_SH_C2_XY = 0.5 * np.sqrt(15.0 / (4 * np.pi))
_SH_C2_Z2 = 0.5 * np.sqrt(5.0 / (16 * np.pi))
_SH_C2_X2Y2 = 0.5 * np.sqrt(15.0 / (16 * np.pi))


def _axis_coords(p):
    i = jnp.minimum(jnp.maximum(p * 15.875 + 63.5, 0.0), 127.0)
    i0 = i.astype(jnp.int32)
    i0 = jnp.where(i0.astype(jnp.float32) > i, i0 - 1, i0)
    w = i - i0.astype(jnp.float32)
    i1 = jnp.minimum(i0 + 1, _GRID - 1)
    return i0, i1, w


def _repack_body(vox_ref, out_ref):
    v = vox_ref[0]
    lo = lax.bitcast_convert_type(v[:_W], jnp.int32)
    hi = lax.bitcast_convert_type(
        jnp.concatenate(
            [v[_W:], jnp.zeros((_CP - _C, _GRID, _GRID), v.dtype)], axis=0),
        jnp.int32)
    lo = lo + 0x7FFF + (lax.shift_right_logical(lo, 16) & 1)
    hi = hi + 0x7FFF + (lax.shift_right_logical(hi, 16) & 1)
    w = lax.shift_right_logical(lo, 16) | (hi & jnp.int32(-65536))
    for y in range(_GRID):
        t = jnp.transpose(w[:, y, :], (1, 0))
        t8 = t.reshape(_W, 8, _W)
        pieces = [t8[:, q, :] for q in range(8)]
        cat = jnp.concatenate(pieces, axis=-1)
        out_ref[0, pl.ds(y * _W, _W), :] = cat


def _tc_repack(voxels_t):
    return pl.pallas_call(
        _repack_body,
        grid=(_GRID,),
        in_specs=[pl.BlockSpec((1, _C, _GRID, _GRID), lambda i: (i, 0, 0, 0))],
        out_specs=pl.BlockSpec(
            (1, _GRID * _GRID * _W // 128, 128), lambda i: (i, 0, 0)),
        out_shape=jax.ShapeDtypeStruct(
            (_GRID, _GRID * _GRID * _W // 128, 128), jnp.int32),
    )(voxels_t)


def _sc_fused(pts3, table):
    nchunks = pts3.shape[1]
    n = nchunks * _B
    pw = n // _NW
    nch = pw // _B
    mesh = plsc.VectorSubcoreMesh(core_axis_name="c", subcore_axis_name="s")

    @functools.partial(
        pl.kernel,
        mesh=mesh,
        compiler_params=pltpu.CompilerParams(use_tc_tiling_on_sc=False),
        out_type=jax.ShapeDtypeStruct((nchunks, _B, _CP), jnp.float32),
        scratch_types=[
            pltpu.VMEM((3, _B), jnp.float32),
            pltpu.VMEM((2, 8, _B), jnp.int32),
            pltpu.VMEM((2, 8, _B), jnp.float32),
            pltpu.VMEM((2, 8, _B, _W), jnp.int32),
            pltpu.VMEM((_B, _CP), jnp.float32),
            pltpu.SemaphoreType.DMA,
            pltpu.SemaphoreType.DMA,
        ],
    )
    def k(pts_hbm, table_hbm, out_hbm, pts_v, idx_v, w_v, rows_v, ft_v,
          sem0, sem1):
        wid = lax.axis_index("s") * 2 + lax.axis_index("c")
        chunk0 = wid * nch
        sems = (sem0, sem1)

        def stage_fire(ci, b):
            for d in range(3):
                pltpu.sync_copy(pts_hbm.at[d, chunk0 + ci], pts_v.at[d])
            for g in range(_B // 16):
                s = g * 16
                x0, x1, wx = _axis_coords(pts_v[0, pl.ds(s, 16)])
                y0, y1, wy = _axis_coords(pts_v[1, pl.ds(s, 16)])
                z0, z1, wz = _axis_coords(pts_v[2, pl.ds(s, 16)])
                ox, oy, oz = 1.0 - wx, 1.0 - wy, 1.0 - wz
                b00 = (z0 * _GRID + y0) * _GRID
                b01 = (z0 * _GRID + y1) * _GRID
                b10 = (z1 * _GRID + y0) * _GRID
                b11 = (z1 * _GRID + y1) * _GRID
                corn = (
                    (b00 + x0, oz * oy * ox), (b00 + x1, oz * oy * wx),
                    (b01 + x0, oz * wy * ox), (b01 + x1, oz * wy * wx),
                    (b10 + x0, wz * oy * ox), (b10 + x1, wz * oy * wx),
                    (b11 + x0, wz * wy * ox), (b11 + x1, wz * wy * wx),
                )
                for c, (iv, wv) in enumerate(corn):
                    idx_v[b, c, pl.ds(s, 16)] = iv
                    w_v[b, c, pl.ds(s, 16)] = wv
            for c in range(8):
                pltpu.async_copy(
                    table_hbm.at[idx_v.at[b, c]], rows_v.at[b, c], sems[b])

        def stage_blend(ci, b):
            for c in range(8):
                pltpu.make_async_copy(
                    table_hbm.at[idx_v.at[b, c]], rows_v.at[b, c],
                    sems[b]).wait()
            for g in range(_B // 16):
                s = g * 16
                wrows = [w_v[b, c, pl.ds(s, 16)] for c in range(8)]

                def ptl(l, carry2, s=s, wrows=wrows):
                    p = s + l
                    lsel = jnp.full((16,), l, jnp.int32)
                    himask = jnp.full((16,), -65536, jnp.int32)
                    acc_lo = acc_hi = None
                    for c in range(8):
                        wb = jnp.take(wrows[c], lsel)
                        pr = rows_v[b, c, p, :]
                        ev = lax.bitcast_convert_type(pr * 65536, jnp.float32)
                        od = lax.bitcast_convert_type(pr & himask, jnp.float32)
                        if acc_lo is None:
                            acc_lo, acc_hi = wb * ev, wb * od
                        else:
                            acc_lo = acc_lo + wb * ev
                            acc_hi = acc_hi + wb * od
                    ft_v[p, pl.ds(0, 16)] = acc_lo
                    ft_v[p, pl.ds(16, 16)] = acc_hi
                    return carry2

                lax.fori_loop(0, 16, ptl, 0)
            pltpu.sync_copy(ft_v, out_hbm.at[chunk0 + ci])

        stage_fire(0, 0)

        def pair(k2, carry):
            ci0 = 2 * k2
            stage_fire(ci0 + 1, 1)
            stage_blend(ci0, 0)

            @pl.when(ci0 + 2 < nch)
            def _():
                stage_fire(ci0 + 2, 0)

            stage_blend(ci0 + 1, 1)
            return carry

        lax.fori_loop(0, nch // 2, pair, 0)

    return k(pts3, table)


def _shade_body(pts_ref, ft_ref, rgb_ref, sig_ref):
    for i in range(8):
        ftt = jnp.transpose(ft_ref[i], (1, 0))
        px = pts_ref[0, i, :]
        py = pts_ref[1, i, :]
        pz = pts_ref[2, i, :]
        rn = lax.rsqrt(px * px + py * py + pz * pz)
        x, y, z = px * rn, py * rn, pz * rn
        basis = [
            _SH_C0 * jnp.ones_like(x),
            _SH_C1 * y, _SH_C1 * z, _SH_C1 * x,
            _SH_C2_XY * (x * y), _SH_C2_XY * (y * z),
            _SH_C2_Z2 * (3.0 * z * z - 1.0),
            _SH_C2_XY * (x * z), _SH_C2_X2Y2 * (x * x - y * y),
        ]
        for r in range(3):
            acc = basis[0] * ftt[1 + 9 * r, :]
            for kk in range(1, 9):
                acc = acc + basis[kk] * ftt[1 + 9 * r + kk, :]
            rgb_ref[r, i, :] = 1.0 / (1.0 + jnp.exp(-acc))
        d = ftt[0, :]
        sig_ref[0, i, :] = (
            jnp.maximum(d, 0.0) + jnp.log1p(jnp.exp(-jnp.abs(d))))


def _tc_shade(pts3, ft):
    nchunks = ft.shape[0]
    bk = 8
    return pl.pallas_call(
        _shade_body,
        grid=(nchunks // bk,),
        in_specs=[
            pl.BlockSpec((3, bk, _B), lambda i: (0, i, 0)),
            pl.BlockSpec((bk, _B, _CP), lambda i: (i, 0, 0)),
        ],
        out_specs=[
            pl.BlockSpec((3, bk, _B), lambda i: (0, i, 0)),
            pl.BlockSpec((1, bk, _B), lambda i: (0, i, 0)),
        ],
        out_shape=[
            jax.ShapeDtypeStruct((3, nchunks, _B), jnp.float32),
            jax.ShapeDtypeStruct((1, nchunks, _B), jnp.float32),
        ],
    )(pts3, ft)


def kernel(points, voxels):
    n = points.shape[0]
    table = _tc_repack(jnp.transpose(voxels, (0, 3, 1, 2))).reshape(-1, _W)
    pts3 = points.T.reshape(3, n // _B, _B)
    ft = _sc_fused(pts3, table)
    rgb3, sig3 = _tc_shade(pts3, ft)
    rgb = rgb3.reshape(3, n).T
    sigma = sig3.reshape(n)[:, None]
    return rgb, sigma
